# Initial kernel scaffold; baseline (speedup 1.0000x reference)
import jax
import jax.numpy as jnp
from jax.experimental import pallas as pl


def kernel(x, A, temperature, q):
    raise NotImplementedError("write your pallas kernel here")


# speedup vs baseline: 9.1235x; 9.1235x over previous
import jax
import jax.numpy as jnp
from jax.experimental import pallas as pl
from jax.experimental.pallas import tpu as pltpu

KTOP = 16


def _fused_kernel(s_ref, xr_ref, xc_ref, q_ref, vals_ref, idx_ref):
    xr = xr_ref[0]
    xc = xc_ref[0]
    q = q_ref[0]
    s = s_ref[0]

    dot = jax.lax.dot_general(
        xr, xc, (((1,), (1,)), ((), ())),
        preferred_element_type=jnp.float32,
        precision=jax.lax.Precision.DEFAULT,
    )
    x2r = jnp.sum(xr * xr, axis=1)[:, None]
    x2c = jnp.sum(xc * xc, axis=1)[None, :]
    d2 = jnp.maximum(x2r + x2c - 2.0 * dot, 0.0)

    score = jnp.log(-jnp.log(q)) - d2 * s

    n = score.shape[1]
    iota = jax.lax.broadcasted_iota(jnp.int32, score.shape, 1)
    cur = score
    vals = []
    idxs = []
    for _ in range(KTOP):
        m = jnp.max(cur, axis=1, keepdims=True)
        sel = cur == m
        ival = jnp.min(jnp.where(sel, iota, n), axis=1, keepdims=True)
        vals.append(m)
        idxs.append(ival)
        cur = jnp.where(iota == ival, -jnp.inf, cur)
    vals_ref[0] = jnp.concatenate(vals, axis=1)
    idx_ref[0] = jnp.concatenate(idxs, axis=1)


def _topk(x, s, q, row_block):
    b, n, d = x.shape
    grid = (b, n // row_block)
    vals, idx = pl.pallas_call(
        _fused_kernel,
        grid=grid,
        in_specs=[
            pl.BlockSpec(memory_space=pltpu.SMEM),
            pl.BlockSpec((1, row_block, d), lambda bi, ri: (bi, ri, 0)),
            pl.BlockSpec((1, n, d), lambda bi, ri: (bi, 0, 0)),
            pl.BlockSpec((1, row_block, n), lambda bi, ri: (bi, ri, 0)),
        ],
        out_specs=[
            pl.BlockSpec((1, row_block, KTOP), lambda bi, ri: (bi, ri, 0)),
            pl.BlockSpec((1, row_block, KTOP), lambda bi, ri: (bi, ri, 0)),
        ],
        out_shape=[
            jax.ShapeDtypeStruct((b, n, KTOP), jnp.float32),
            jax.ShapeDtypeStruct((b, n, KTOP), jnp.int32),
        ],
    )(s, x, x, q)
    return vals, idx


def kernel(x, A, temperature, q):
    b, n, d = x.shape
    s = jnp.exp(jnp.clip(temperature, -5.0, 5.0)).reshape(1)
    logprobs, indices = _topk(x, s, q, 256)

    rows = jnp.broadcast_to(
        jnp.arange(n, dtype=indices.dtype)[None, :, None], (b, n, KTOP)
    )
    edges = jnp.stack((indices.reshape(b, -1), rows.reshape(b, -1)), axis=-2)
    offset = (jnp.arange(b, dtype=indices.dtype) * n)[:, None, None]
    edges_hat = jnp.transpose(edges + offset, (1, 0, 2)).reshape(2, -1)
    return (x, edges_hat, logprobs)
